# baseline (speedup 1.0000x reference)
import functools

import jax
import jax.numpy as jnp
from jax import lax
from jax.experimental import pallas as pl
from jax.experimental.pallas import tpu as pltpu
from jax.experimental.pallas import tpu_sc as plsc

_EMBED_DIM = 128
_BATCH = 16384
_NC, _NS = 2, 16
_NW = _NC * _NS
_BPW = _BATCH // _NW
_CS = 128
_NCHUNK = _BPW // _CS

_mesh = plsc.VectorSubcoreMesh(core_axis_name="c", subcore_axis_name="s")


@functools.partial(
    pl.kernel,
    mesh=_mesh,
    out_type=jax.ShapeDtypeStruct((_BATCH, _EMBED_DIM), jnp.float32),
    scratch_types=[
        pltpu.VMEM((_NCHUNK, _CS), jnp.int32),
        pltpu.VMEM((_BPW, _EMBED_DIM), jnp.float32),
        pltpu.SemaphoreType.DMA,
        pltpu.SemaphoreType.DMA,
        pltpu.SemaphoreType.DMA,
        pltpu.SemaphoreType.DMA,
        pltpu.SemaphoreType.DMA,
    ],
)
def _embed(labels_hbm, table_hbm, out_hbm, idx_v, rows_v, wsem, g0, g1, g2, g3):
    wid = lax.axis_index("s") * _NC + lax.axis_index("c")
    pltpu.sync_copy(labels_hbm.at[pl.ds(wid * _NCHUNK, _NCHUNK)], idx_v)
    gsems = (g0, g1, g2, g3)
    gathers = [
        pltpu.async_copy(
            table_hbm.at[idx_v.at[j]],
            rows_v.at[pl.ds(j * _CS, _CS)],
            gsems[j],
        )
        for j in range(_NCHUNK)
    ]
    writes = []
    for j in range(_NCHUNK):
        gathers[j].wait()
        writes.append(
            pltpu.async_copy(
                rows_v.at[pl.ds(j * _CS, _CS)],
                out_hbm.at[pl.ds(wid * _BPW + j * _CS, _CS)],
                wsem,
            )
        )
    for w in writes:
        w.wait()


def kernel(labels, train, embedding_table):
    del train
    idx = labels.astype(jnp.int32).reshape(_NW * _NCHUNK, _CS)
    return _embed(idx, embedding_table)


# speedup vs baseline: 1.1581x; 1.1581x over previous
import functools

import jax
import jax.numpy as jnp
from jax import lax
from jax.experimental import pallas as pl
from jax.experimental.pallas import tpu as pltpu
from jax.experimental.pallas import tpu_sc as plsc

_ROWS = 1001
_EMBED_DIM = 128
_BATCH = 16384
_NC, _NS = 2, 16
_NW = _NC * _NS
_BPW = _BATCH // _NW
_CS = 128
_NCHUNK = _BPW // _CS
_STG = 64

_mesh = plsc.VectorSubcoreMesh(core_axis_name="c", subcore_axis_name="s")


@functools.partial(
    pl.kernel,
    mesh=_mesh,
    out_type=jax.ShapeDtypeStruct((_BATCH, _EMBED_DIM), jnp.float32),
    scratch_types=[
        pltpu.VMEM((_NCHUNK, _CS), jnp.int32),
        pltpu.VMEM((_BPW, _EMBED_DIM), jnp.float32),
        pltpu.VMEM_SHARED((_ROWS, _EMBED_DIM), jnp.float32),
        pltpu.SemaphoreType.DMA,
        pltpu.SemaphoreType.DMA,
        pltpu.SemaphoreType.DMA,
        pltpu.SemaphoreType.DMA,
        pltpu.SemaphoreType.DMA,
        pltpu.SemaphoreType.DMA,
    ],
)
def _embed(labels_hbm, table_hbm, out_hbm, idx_v, rows_v, tbl_s,
           ssem, wsem, g0, g1, g2, g3):
    sid = lax.axis_index("s")
    wid = sid * _NC + lax.axis_index("c")

    @pl.when(sid < _NS - 1)
    def _():
        pltpu.async_copy(table_hbm.at[pl.ds(sid * _STG, _STG)],
                         tbl_s.at[pl.ds(sid * _STG, _STG)], ssem).wait()

    @pl.when(sid == _NS - 1)
    def _():
        tail = _ROWS - (_NS - 1) * _STG
        pltpu.async_copy(table_hbm.at[pl.ds((_NS - 1) * _STG, tail)],
                         tbl_s.at[pl.ds((_NS - 1) * _STG, tail)], ssem).wait()

    pltpu.sync_copy(labels_hbm.at[pl.ds(wid * _NCHUNK, _NCHUNK)], idx_v)
    plsc.subcore_barrier()

    gsems = (g0, g1, g2, g3)
    gathers = [
        pltpu.async_copy(
            tbl_s.at[idx_v.at[j]],
            rows_v.at[pl.ds(j * _CS, _CS)],
            gsems[j],
        )
        for j in range(_NCHUNK)
    ]
    writes = []
    for j in range(_NCHUNK):
        gathers[j].wait()
        writes.append(
            pltpu.async_copy(
                rows_v.at[pl.ds(j * _CS, _CS)],
                out_hbm.at[pl.ds(wid * _BPW + j * _CS, _CS)],
                wsem,
            )
        )
    for w in writes:
        w.wait()


def kernel(labels, train, embedding_table):
    del train
    idx = labels.astype(jnp.int32).reshape(_NW * _NCHUNK, _CS)
    return _embed(idx, embedding_table)


# speedup vs baseline: 1.1694x; 1.0097x over previous
import functools

import jax
import jax.numpy as jnp
from jax import lax
from jax.experimental import pallas as pl
from jax.experimental.pallas import tpu as pltpu
from jax.experimental.pallas import tpu_sc as plsc

_ROWS = 1001
_EMBED_DIM = 128
_BATCH = 16384
_NC, _NS = 2, 16
_NW = _NC * _NS
_BPW = _BATCH // _NW
_CS = 64
_NCHUNK = _BPW // _CS
_STG = 64

_mesh = plsc.VectorSubcoreMesh(core_axis_name="c", subcore_axis_name="s")


@functools.partial(
    pl.kernel,
    mesh=_mesh,
    out_type=jax.ShapeDtypeStruct((_BATCH, _EMBED_DIM), jnp.float32),
    scratch_types=[
        pltpu.VMEM((_NCHUNK, _CS), jnp.int32),
        pltpu.VMEM((_BPW, _EMBED_DIM), jnp.float32),
        pltpu.VMEM_SHARED((_ROWS, _EMBED_DIM), jnp.float32),
        pltpu.SemaphoreType.DMA,
        pltpu.SemaphoreType.DMA,
    ] + [pltpu.SemaphoreType.DMA] * _NCHUNK,
)
def _embed(labels_hbm, table_hbm, out_hbm, idx_v, rows_v, tbl_s,
           ssem, wsem, *gsems):
    sid = lax.axis_index("s")
    wid = sid * _NC + lax.axis_index("c")

    @pl.when(sid < _NS - 1)
    def _():
        pltpu.async_copy(table_hbm.at[pl.ds(sid * _STG, _STG)],
                         tbl_s.at[pl.ds(sid * _STG, _STG)], ssem).wait()

    @pl.when(sid == _NS - 1)
    def _():
        tail = _ROWS - (_NS - 1) * _STG
        pltpu.async_copy(table_hbm.at[pl.ds((_NS - 1) * _STG, tail)],
                         tbl_s.at[pl.ds((_NS - 1) * _STG, tail)], ssem).wait()

    pltpu.sync_copy(labels_hbm.at[pl.ds(wid * _NCHUNK, _NCHUNK)], idx_v)
    plsc.subcore_barrier()

    gathers = [
        pltpu.async_copy(
            tbl_s.at[idx_v.at[j]],
            rows_v.at[pl.ds(j * _CS, _CS)],
            gsems[j],
        )
        for j in range(_NCHUNK)
    ]
    writes = []
    for j in range(_NCHUNK):
        gathers[j].wait()
        writes.append(
            pltpu.async_copy(
                rows_v.at[pl.ds(j * _CS, _CS)],
                out_hbm.at[pl.ds(wid * _BPW + j * _CS, _CS)],
                wsem,
            )
        )
    for w in writes:
        w.wait()


def kernel(labels, train, embedding_table):
    del train
    idx = labels.astype(jnp.int32).reshape(_NW * _NCHUNK, _CS)
    return _embed(idx, embedding_table)
